# baseline (speedup 1.0000x reference)
import functools

import jax
import jax.numpy as jnp
from jax import lax
from jax.experimental import pallas as pl
from jax.experimental.pallas import tpu as pltpu
from jax.experimental.pallas import tpu_sc as plsc

N_NODES = 10000
N_PAD = 10240
N_EDGES = 320000
D = 128
H = 64
NC = 2
NS = 16
NW = NC * NS

DEG_E_PER_TILE = N_EDGES // NW
DEG_CHUNKS = DEG_E_PER_TILE // 16
DEG_PAD = 10240
DEG_ROWS = DEG_PAD // 16
DEG_RED_BATCHES = DEG_ROWS // 128

PROP_BATCH = 128
CHUNK = 4
TOTC = 40
NCH0 = 32
NCH1 = TOTC - NCH0
E_PAD = NS * TOTC * CHUNK * PROP_BATCH
ROWS_PER_TILE = N_PAD // NS

_MESH = plsc.VectorSubcoreMesh(core_axis_name="c", subcore_axis_name="s")
_SC_PARAMS = pltpu.CompilerParams(needs_layout_passes=False)


DEG_SLICE = DEG_PAD // NS


def _deg_body(dst_hbm, ew_hbm, deg_hbm, dstv, ewv, degv, redv, outv, shared_parts):
    c = lax.axis_index("c")
    s = lax.axis_index("s")
    wid = s * NC + c
    pltpu.sync_copy(dst_hbm.at[wid, 0], dstv)
    pltpu.sync_copy(ew_hbm.at[wid, 0], ewv)

    zero16 = jnp.zeros((16,), jnp.float32)

    def zero_chunk(i, _):
        degv[pl.ds(i * 16, 16)] = zero16
        return 0

    lax.fori_loop(0, DEG_ROWS, zero_chunk, 0)

    def acc(i, _):
        d16 = dstv[pl.ds(i * 16, 16)]
        w16 = ewv[pl.ds(i * 16, 16)]
        plsc.addupdate_scatter(degv, [d16], w16)
        return 0

    lax.fori_loop(0, DEG_CHUNKS, acc, 0)

    pltpu.sync_copy(degv, shared_parts.at[s])
    plsc.subcore_barrier()

    for t in range(NS):
        pltpu.sync_copy(shared_parts.at[t, pl.ds(s * DEG_SLICE, DEG_SLICE)],
                        redv.at[t])

    def red(k, _):
        acc16 = redv[0, pl.ds(k * 16, 16)]
        for t in range(1, NS):
            acc16 = acc16 + redv[t, pl.ds(k * 16, 16)]
        outv[pl.ds(k * 16, 16)] = acc16
        return 0

    lax.fori_loop(0, DEG_SLICE // 16, red, 0)
    pltpu.sync_copy(outv, deg_hbm.at[c, 0, pl.ds(s * DEG_SLICE, DEG_SLICE)])


_sc_deg = pl.kernel(
    _deg_body,
    out_type=jax.ShapeDtypeStruct((NC, 1, DEG_PAD), jnp.float32),
    mesh=_MESH,
    scratch_types=[
        pltpu.VMEM((DEG_E_PER_TILE,), jnp.int32),
        pltpu.VMEM((DEG_E_PER_TILE,), jnp.float32),
        pltpu.VMEM((DEG_PAD,), jnp.float32),
        pltpu.VMEM((NS, DEG_SLICE), jnp.float32),
        pltpu.VMEM((DEG_SLICE,), jnp.float32),
        pltpu.VMEM_SHARED((NS, DEG_PAD), jnp.float32),
    ],
    compiler_params=_SC_PARAMS,
)


def _tc1_body(x_ref, w_ref, deg_ref, h_ref):
    deg = deg_ref[:, :1] + deg_ref[:, 1:2] + 1.0
    dinv = jnp.where(deg > 0, lax.rsqrt(deg), 0.0)
    h = lax.dot_general(x_ref[...], w_ref[...],
                        (((1,), (1,)), ((), ())),
                        preferred_element_type=jnp.float32)
    h_ref[...] = h * dinv


def _tc1(x, W, deg2):
    R = 2048
    return pl.pallas_call(
        _tc1_body,
        grid=(N_PAD // R,),
        in_specs=[
            pl.BlockSpec((R, D), lambda i: (i, 0)),
            pl.BlockSpec((D, D), lambda i: (0, 0)),
            pl.BlockSpec((R, 2), lambda i: (i, 0)),
        ],
        out_specs=pl.BlockSpec((R, D), lambda i: (i, 0)),
        out_shape=jax.ShapeDtypeStruct((N_PAD, D), jnp.float32),
    )(x, W, deg2)


def _prop_body(h_hbm, src_hbm, dst_hbm, ew_hbm, out_hbm,
               src_a, dst_a, ew_a, src_b, dst_b, ew_b,
               rows_0, rows_1, shared_o,
               ssem_a, ssem_b, rsem_0, rsem_1, wsem):
    c = lax.axis_index("c")
    s = lax.axis_index("s")
    rbase = s * ROWS_PER_TILE

    zero16 = jnp.zeros((16,), jnp.float32)
    stg = ((src_a, dst_a, ew_a, ssem_a), (src_b, dst_b, ew_b, ssem_b))
    rows = (rows_0, rows_1)
    rsems = (rsem_0, rsem_1)

    base = jnp.where(c == 0, 0, NCH0)
    nch = jnp.where(c == 0, NCH0, NCH1)

    def stage_async(q, bufs):
        sc_, dc_, ec_, sem_ = bufs
        pltpu.async_copy(src_hbm.at[s, base + q], sc_, sem_)
        pltpu.async_copy(dst_hbm.at[s, base + q], dc_, sem_)
        pltpu.async_copy(ew_hbm.at[s, base + q, 0], ec_, sem_)

    def stage_wait(bufs):
        sc_, dc_, ec_, sem_ = bufs
        pltpu.make_async_copy(src_hbm.at[s, 0], sc_, sem_).wait()
        pltpu.make_async_copy(dst_hbm.at[s, 0], dc_, sem_).wait()
        pltpu.make_async_copy(ew_hbm.at[s, 0, 0], ec_, sem_).wait()

    @pl.when(c == 0)
    def _():
        pltpu.sync_copy(h_hbm.at[pl.ds(rbase, ROWS_PER_TILE)],
                        shared_o.at[pl.ds(rbase, ROWS_PER_TILE)])

    @pl.when(c == 1)
    def _():
        def zrow(i, _):
            for f in range(D // 16):
                rows_0[i, pl.ds(f * 16, 16)] = zero16
            return 0

        lax.fori_loop(0, PROP_BATCH, zrow, 0)
        for z in range(ROWS_PER_TILE // PROP_BATCH):
            pltpu.sync_copy(rows_0,
                            shared_o.at[pl.ds(rbase + z * PROP_BATCH,
                                              PROP_BATCH)])

    plsc.subcore_barrier()

    pltpu.sync_copy(src_hbm.at[s, base], src_a)
    pltpu.sync_copy(dst_hbm.at[s, base], dst_a)
    pltpu.sync_copy(ew_hbm.at[s, base, 0], ew_a)
    stage_async(1, stg[1])
    pltpu.async_copy(h_hbm.at[src_a.at[0]], rows_0, rsem_0)

    def superpair(qq, _):
        for p in range(2):
            q = qq * 2 + p
            sc_, dc_, ec_, _sem = stg[p]
            nstg = stg[1 - p]
            for j in range(CHUNK):
                rb, rsem = rows[j % 2], rsems[j % 2]
                nb, nsem = rows[(j + 1) % 2], rsems[(j + 1) % 2]
                pltpu.make_async_copy(h_hbm.at[sc_.at[j]], rb, rsem).wait()
                if j == 0:
                    @pl.when(q > 0)
                    def _():
                        pltpu.make_async_copy(
                            nb, shared_o.at[dc_.at[0]], wsem).wait()
                else:
                    pltpu.make_async_copy(
                        nb, shared_o.at[dc_.at[j - 1]], wsem).wait()
                if j < CHUNK - 1:
                    pltpu.async_copy(h_hbm.at[sc_.at[j + 1]], nb, nsem)
                else:
                    @pl.when(q + 1 < nch)
                    def _():
                        stage_wait(nstg)
                        pltpu.async_copy(h_hbm.at[nstg[0].at[0]], nb, nsem)

                def scale(i, _):
                    r = i * 2
                    bc0 = plsc.load_gather(
                        ec_, [jnp.full((16,), j * PROP_BATCH + r, jnp.int32)])
                    bc1 = plsc.load_gather(
                        ec_, [jnp.full((16,), j * PROP_BATCH + r + 1,
                                       jnp.int32)])
                    for f in range(D // 16):
                        rb[r, pl.ds(f * 16, 16)] = (
                            rb[r, pl.ds(f * 16, 16)] * bc0)
                    for f in range(D // 16):
                        rb[r + 1, pl.ds(f * 16, 16)] = (
                            rb[r + 1, pl.ds(f * 16, 16)] * bc1)
                    return 0

                lax.fori_loop(0, PROP_BATCH // 2, scale, 0)
                pltpu.async_copy(rb, shared_o.at[dc_.at[j]], wsem, add=True)

            @pl.when(q + 2 < nch)
            def _():
                stage_async(q + 2, (sc_, dc_, ec_, _sem))

        return 0

    lax.fori_loop(0, nch // 2, superpair, 0)
    pltpu.make_async_copy(rows_1, shared_o.at[dst_a.at[0]], wsem).wait()
    plsc.subcore_barrier()
    pltpu.sync_copy(shared_o.at[pl.ds(rbase, ROWS_PER_TILE)],
                    out_hbm.at[c, pl.ds(rbase, ROWS_PER_TILE)])


_sc_prop = pl.kernel(
    _prop_body,
    out_type=jax.ShapeDtypeStruct((NC, N_PAD, D), jnp.float32),
    mesh=_MESH,
    scratch_types=[
        pltpu.VMEM((CHUNK, PROP_BATCH), jnp.int32),
        pltpu.VMEM((CHUNK, PROP_BATCH), jnp.int32),
        pltpu.VMEM((CHUNK * PROP_BATCH,), jnp.float32),
        pltpu.VMEM((CHUNK, PROP_BATCH), jnp.int32),
        pltpu.VMEM((CHUNK, PROP_BATCH), jnp.int32),
        pltpu.VMEM((CHUNK * PROP_BATCH,), jnp.float32),
        pltpu.VMEM((PROP_BATCH, D), jnp.float32),
        pltpu.VMEM((PROP_BATCH, D), jnp.float32),
        pltpu.VMEM_SHARED((N_PAD, D), jnp.float32),
        pltpu.SemaphoreType.DMA,
        pltpu.SemaphoreType.DMA,
        pltpu.SemaphoreType.DMA,
        pltpu.SemaphoreType.DMA,
        pltpu.SemaphoreType.DMA,
    ],
    compiler_params=_SC_PARAMS,
)


def _tc2_body(o_ref, deg_ref, b_ref, ls_ref, lb_ref, out_ref):
    deg = deg_ref[:, :1] + deg_ref[:, 1:2] + 1.0
    dinv = jnp.where(deg > 0, lax.rsqrt(deg), 0.0)
    o = o_ref[0] + o_ref[1]
    o = o * dinv + b_ref[...]
    mean = jnp.mean(o, axis=1, keepdims=True)
    cent = o - mean
    var = jnp.mean(cent * cent, axis=1, keepdims=True)
    o = cent * lax.rsqrt(var + 1e-5) * ls_ref[...] + lb_ref[...]
    out_ref[...] = jnp.maximum(o, 0.0)


def _tc2(o2, deg2, b, ls, lb):
    R = 2048
    return pl.pallas_call(
        _tc2_body,
        grid=(N_PAD // R,),
        in_specs=[
            pl.BlockSpec((2, R, D), lambda i: (0, i, 0)),
            pl.BlockSpec((R, 2), lambda i: (i, 0)),
            pl.BlockSpec((1, D), lambda i: (0, 0)),
            pl.BlockSpec((1, D), lambda i: (0, 0)),
            pl.BlockSpec((1, D), lambda i: (0, 0)),
        ],
        out_specs=pl.BlockSpec((R, D), lambda i: (i, 0)),
        out_shape=jax.ShapeDtypeStruct((N_PAD, D), jnp.float32),
    )(o2, deg2, b, ls, lb)


@jax.jit
def kernel(x, edge_index, edge_weight, W, b, ln_scale, ln_bias):
    ei = edge_index.astype(jnp.int32)
    src = ei[0]
    dst = ei[1]
    ew = edge_weight.astype(jnp.float32)

    dst1 = dst.reshape(NW, 1, DEG_E_PER_TILE)
    ew1 = ew.reshape(NW, 1, DEG_E_PER_TILE)
    deg_parts = _sc_deg(dst1, ew1)
    deg2 = deg_parts.reshape(NC, DEG_PAD).T

    xp = jnp.pad(x, ((0, N_PAD - N_NODES), (0, 0)))
    h = _tc1(xp, W, deg2)

    pad = E_PAD - N_EDGES
    zi = jnp.zeros((pad,), jnp.int32)
    zf = jnp.zeros((pad,), jnp.float32)
    srcp = jnp.concatenate([src, zi]).reshape(NS, TOTC, CHUNK, PROP_BATCH)
    dstp = jnp.concatenate([dst, zi]).reshape(NS, TOTC, CHUNK, PROP_BATCH)
    ewp = jnp.concatenate([ew, zf]).reshape(NS, TOTC, 1, CHUNK * PROP_BATCH)
    o2 = _sc_prop(h, srcp, dstp, ewp)

    out = _tc2(o2, deg2, b.reshape(1, D), ln_scale.reshape(1, D),
               ln_bias.reshape(1, D))
    return out[:N_NODES]


# speedup vs baseline: 1.0421x; 1.0421x over previous
import functools

import jax
import jax.numpy as jnp
from jax import lax
from jax.experimental import pallas as pl
from jax.experimental.pallas import tpu as pltpu
from jax.experimental.pallas import tpu_sc as plsc

N_NODES = 10000
N_PAD = 10240
N_EDGES = 320000
D = 128
H = 64
NC = 2
NS = 16
NW = NC * NS

DEG_E_PER_TILE = N_EDGES // NW
DEG_CHUNKS = DEG_E_PER_TILE // 16
DEG_PAD = 10240
DEG_ROWS = DEG_PAD // 16
DEG_RED_BATCHES = DEG_ROWS // 128

PROP_BATCH = 128
CHUNK = 4
TOTC = 40
NCH0 = 30
NCH1 = TOTC - NCH0
E_PAD = NS * TOTC * CHUNK * PROP_BATCH
ROWS_PER_TILE = N_PAD // NS

_MESH = plsc.VectorSubcoreMesh(core_axis_name="c", subcore_axis_name="s")
_SC_PARAMS = pltpu.CompilerParams(needs_layout_passes=False)


DEG_SLICE = DEG_PAD // NS


def _deg_body(dst_hbm, ew_hbm, deg_hbm, dstv, ewv, degv, redv, outv, shared_parts):
    c = lax.axis_index("c")
    s = lax.axis_index("s")
    wid = s * NC + c
    pltpu.sync_copy(dst_hbm.at[wid, 0], dstv)
    pltpu.sync_copy(ew_hbm.at[wid, 0], ewv)

    zero16 = jnp.zeros((16,), jnp.float32)

    def zero_chunk(i, _):
        degv[pl.ds(i * 16, 16)] = zero16
        return 0

    lax.fori_loop(0, DEG_ROWS, zero_chunk, 0)

    def acc(i, _):
        d16 = dstv[pl.ds(i * 16, 16)]
        w16 = ewv[pl.ds(i * 16, 16)]
        plsc.addupdate_scatter(degv, [d16], w16)
        return 0

    lax.fori_loop(0, DEG_CHUNKS, acc, 0)

    pltpu.sync_copy(degv, shared_parts.at[s])
    plsc.subcore_barrier()

    for t in range(NS):
        pltpu.sync_copy(shared_parts.at[t, pl.ds(s * DEG_SLICE, DEG_SLICE)],
                        redv.at[t])

    def red(k, _):
        acc16 = redv[0, pl.ds(k * 16, 16)]
        for t in range(1, NS):
            acc16 = acc16 + redv[t, pl.ds(k * 16, 16)]
        outv[pl.ds(k * 16, 16)] = acc16
        return 0

    lax.fori_loop(0, DEG_SLICE // 16, red, 0)
    pltpu.sync_copy(outv, deg_hbm.at[c, 0, pl.ds(s * DEG_SLICE, DEG_SLICE)])


_sc_deg = pl.kernel(
    _deg_body,
    out_type=jax.ShapeDtypeStruct((NC, 1, DEG_PAD), jnp.float32),
    mesh=_MESH,
    scratch_types=[
        pltpu.VMEM((DEG_E_PER_TILE,), jnp.int32),
        pltpu.VMEM((DEG_E_PER_TILE,), jnp.float32),
        pltpu.VMEM((DEG_PAD,), jnp.float32),
        pltpu.VMEM((NS, DEG_SLICE), jnp.float32),
        pltpu.VMEM((DEG_SLICE,), jnp.float32),
        pltpu.VMEM_SHARED((NS, DEG_PAD), jnp.float32),
    ],
    compiler_params=_SC_PARAMS,
)


def _tc1_body(x_ref, w_ref, deg_ref, h_ref):
    deg = deg_ref[:, :1] + deg_ref[:, 1:2] + 1.0
    dinv = jnp.where(deg > 0, lax.rsqrt(deg), 0.0)
    h = lax.dot_general(x_ref[...], w_ref[...],
                        (((1,), (1,)), ((), ())),
                        preferred_element_type=jnp.float32)
    h_ref[...] = h * dinv


def _tc1(x, W, deg2):
    R = 2048
    return pl.pallas_call(
        _tc1_body,
        grid=(N_PAD // R,),
        in_specs=[
            pl.BlockSpec((R, D), lambda i: (i, 0)),
            pl.BlockSpec((D, D), lambda i: (0, 0)),
            pl.BlockSpec((R, 2), lambda i: (i, 0)),
        ],
        out_specs=pl.BlockSpec((R, D), lambda i: (i, 0)),
        out_shape=jax.ShapeDtypeStruct((N_PAD, D), jnp.float32),
    )(x, W, deg2)


def _prop_body(h_hbm, src_hbm, dst_hbm, ew_hbm, out_hbm,
               src_a, dst_a, ew_a, src_b, dst_b, ew_b,
               rows_0, rows_1, shared_o,
               ssem_a, ssem_b, rsem_0, rsem_1, wsem):
    c = lax.axis_index("c")
    s = lax.axis_index("s")
    rbase = s * ROWS_PER_TILE

    zero16 = jnp.zeros((16,), jnp.float32)
    stg = ((src_a, dst_a, ew_a, ssem_a), (src_b, dst_b, ew_b, ssem_b))
    rows = (rows_0, rows_1)
    rsems = (rsem_0, rsem_1)

    base = jnp.where(c == 0, 0, NCH0)
    nch = jnp.where(c == 0, NCH0, NCH1)

    def stage_async(q, bufs):
        sc_, dc_, ec_, sem_ = bufs
        pltpu.async_copy(src_hbm.at[s, base + q], sc_, sem_)
        pltpu.async_copy(dst_hbm.at[s, base + q], dc_, sem_)
        pltpu.async_copy(ew_hbm.at[s, base + q, 0], ec_, sem_)

    def stage_wait(bufs):
        sc_, dc_, ec_, sem_ = bufs
        pltpu.make_async_copy(src_hbm.at[s, 0], sc_, sem_).wait()
        pltpu.make_async_copy(dst_hbm.at[s, 0], dc_, sem_).wait()
        pltpu.make_async_copy(ew_hbm.at[s, 0, 0], ec_, sem_).wait()

    @pl.when(c == 0)
    def _():
        pltpu.sync_copy(h_hbm.at[pl.ds(rbase, ROWS_PER_TILE)],
                        shared_o.at[pl.ds(rbase, ROWS_PER_TILE)])

    @pl.when(c == 1)
    def _():
        def zrow(i, _):
            for f in range(D // 16):
                rows_0[i, pl.ds(f * 16, 16)] = zero16
            return 0

        lax.fori_loop(0, PROP_BATCH, zrow, 0)
        for z in range(ROWS_PER_TILE // PROP_BATCH):
            pltpu.sync_copy(rows_0,
                            shared_o.at[pl.ds(rbase + z * PROP_BATCH,
                                              PROP_BATCH)])

    plsc.subcore_barrier()

    pltpu.sync_copy(src_hbm.at[s, base], src_a)
    pltpu.sync_copy(dst_hbm.at[s, base], dst_a)
    pltpu.sync_copy(ew_hbm.at[s, base, 0], ew_a)
    stage_async(1, stg[1])
    pltpu.async_copy(h_hbm.at[src_a.at[0]], rows_0, rsem_0)

    def superpair(qq, _):
        for p in range(2):
            q = qq * 2 + p
            sc_, dc_, ec_, _sem = stg[p]
            nstg = stg[1 - p]
            for j in range(CHUNK):
                rb, rsem = rows[j % 2], rsems[j % 2]
                nb, nsem = rows[(j + 1) % 2], rsems[(j + 1) % 2]
                pltpu.make_async_copy(h_hbm.at[sc_.at[j]], rb, rsem).wait()
                if j == 0:
                    @pl.when(q > 0)
                    def _():
                        pltpu.make_async_copy(
                            nb, shared_o.at[dc_.at[0]], wsem).wait()
                else:
                    pltpu.make_async_copy(
                        nb, shared_o.at[dc_.at[j - 1]], wsem).wait()
                if j < CHUNK - 1:
                    pltpu.async_copy(h_hbm.at[sc_.at[j + 1]], nb, nsem)
                else:
                    @pl.when(q + 1 < nch)
                    def _():
                        stage_wait(nstg)
                        pltpu.async_copy(h_hbm.at[nstg[0].at[0]], nb, nsem)

                def scale(i, _):
                    r = i * 2
                    bc0 = plsc.load_gather(
                        ec_, [jnp.full((16,), j * PROP_BATCH + r, jnp.int32)])
                    bc1 = plsc.load_gather(
                        ec_, [jnp.full((16,), j * PROP_BATCH + r + 1,
                                       jnp.int32)])
                    for f in range(D // 16):
                        rb[r, pl.ds(f * 16, 16)] = (
                            rb[r, pl.ds(f * 16, 16)] * bc0)
                    for f in range(D // 16):
                        rb[r + 1, pl.ds(f * 16, 16)] = (
                            rb[r + 1, pl.ds(f * 16, 16)] * bc1)
                    return 0

                lax.fori_loop(0, PROP_BATCH // 2, scale, 0)
                pltpu.async_copy(rb, shared_o.at[dc_.at[j]], wsem, add=True)

            @pl.when(q + 2 < nch)
            def _():
                stage_async(q + 2, (sc_, dc_, ec_, _sem))

        return 0

    lax.fori_loop(0, nch // 2, superpair, 0)
    pltpu.make_async_copy(rows_1, shared_o.at[dst_a.at[0]], wsem).wait()
    plsc.subcore_barrier()
    pltpu.sync_copy(shared_o.at[pl.ds(rbase, ROWS_PER_TILE)],
                    out_hbm.at[c, pl.ds(rbase, ROWS_PER_TILE)])


_sc_prop = pl.kernel(
    _prop_body,
    out_type=jax.ShapeDtypeStruct((NC, N_PAD, D), jnp.float32),
    mesh=_MESH,
    scratch_types=[
        pltpu.VMEM((CHUNK, PROP_BATCH), jnp.int32),
        pltpu.VMEM((CHUNK, PROP_BATCH), jnp.int32),
        pltpu.VMEM((CHUNK * PROP_BATCH,), jnp.float32),
        pltpu.VMEM((CHUNK, PROP_BATCH), jnp.int32),
        pltpu.VMEM((CHUNK, PROP_BATCH), jnp.int32),
        pltpu.VMEM((CHUNK * PROP_BATCH,), jnp.float32),
        pltpu.VMEM((PROP_BATCH, D), jnp.float32),
        pltpu.VMEM((PROP_BATCH, D), jnp.float32),
        pltpu.VMEM_SHARED((N_PAD, D), jnp.float32),
        pltpu.SemaphoreType.DMA,
        pltpu.SemaphoreType.DMA,
        pltpu.SemaphoreType.DMA,
        pltpu.SemaphoreType.DMA,
        pltpu.SemaphoreType.DMA,
    ],
    compiler_params=_SC_PARAMS,
)


def _tc2_body(o_ref, deg_ref, b_ref, ls_ref, lb_ref, out_ref):
    deg = deg_ref[:, :1] + deg_ref[:, 1:2] + 1.0
    dinv = jnp.where(deg > 0, lax.rsqrt(deg), 0.0)
    o = o_ref[0] + o_ref[1]
    o = o * dinv + b_ref[...]
    mean = jnp.mean(o, axis=1, keepdims=True)
    cent = o - mean
    var = jnp.mean(cent * cent, axis=1, keepdims=True)
    o = cent * lax.rsqrt(var + 1e-5) * ls_ref[...] + lb_ref[...]
    out_ref[...] = jnp.maximum(o, 0.0)


def _tc2(o2, deg2, b, ls, lb):
    R = 2048
    return pl.pallas_call(
        _tc2_body,
        grid=(N_PAD // R,),
        in_specs=[
            pl.BlockSpec((2, R, D), lambda i: (0, i, 0)),
            pl.BlockSpec((R, 2), lambda i: (i, 0)),
            pl.BlockSpec((1, D), lambda i: (0, 0)),
            pl.BlockSpec((1, D), lambda i: (0, 0)),
            pl.BlockSpec((1, D), lambda i: (0, 0)),
        ],
        out_specs=pl.BlockSpec((R, D), lambda i: (i, 0)),
        out_shape=jax.ShapeDtypeStruct((N_PAD, D), jnp.float32),
    )(o2, deg2, b, ls, lb)


@jax.jit
def kernel(x, edge_index, edge_weight, W, b, ln_scale, ln_bias):
    ei = edge_index.astype(jnp.int32)
    src = ei[0]
    dst = ei[1]
    ew = edge_weight.astype(jnp.float32)

    dst1 = dst.reshape(NW, 1, DEG_E_PER_TILE)
    ew1 = ew.reshape(NW, 1, DEG_E_PER_TILE)
    deg_parts = _sc_deg(dst1, ew1)
    deg2 = deg_parts.reshape(NC, DEG_PAD).T

    xp = jnp.pad(x, ((0, N_PAD - N_NODES), (0, 0)))
    h = _tc1(xp, W, deg2)

    pad = E_PAD - N_EDGES
    zi = jnp.zeros((pad,), jnp.int32)
    zf = jnp.zeros((pad,), jnp.float32)
    srcp = jnp.concatenate([src, zi]).reshape(NS, TOTC, CHUNK, PROP_BATCH)
    dstp = jnp.concatenate([dst, zi]).reshape(NS, TOTC, CHUNK, PROP_BATCH)
    ewp = jnp.concatenate([ew, zf]).reshape(NS, TOTC, 1, CHUNK * PROP_BATCH)
    o2 = _sc_prop(h, srcp, dstp, ewp)

    out = _tc2(o2, deg2, b.reshape(1, D), ln_scale.reshape(1, D),
               ln_bias.reshape(1, D))
    return out[:N_NODES]


# speedup vs baseline: 1.0634x; 1.0204x over previous
import functools

import jax
import jax.numpy as jnp
from jax import lax
from jax.experimental import pallas as pl
from jax.experimental.pallas import tpu as pltpu
from jax.experimental.pallas import tpu_sc as plsc

N_NODES = 10000
N_PAD = 10240
N_EDGES = 320000
D = 128
H = 64
NC = 2
NS = 16
NW = NC * NS

DEG_E_PER_TILE = N_EDGES // NW
DEG_CHUNKS = DEG_E_PER_TILE // 16
DEG_PAD = 10240
DEG_ROWS = DEG_PAD // 16
DEG_RED_BATCHES = DEG_ROWS // 128

PROP_BATCH = 128
CHUNK = 4
TOTC = 40
NCH0 = 30
NCH1 = TOTC - NCH0
E_PAD = NS * TOTC * CHUNK * PROP_BATCH
ROWS_PER_TILE = N_PAD // NS

_MESH = plsc.VectorSubcoreMesh(core_axis_name="c", subcore_axis_name="s")
_SC_PARAMS = pltpu.CompilerParams(needs_layout_passes=False)


DEG_SLICE = DEG_PAD // NS


def _deg_body(ei_hbm, ew_hbm, deg_hbm, dstv, ewv, degv, redv, outv, shared_parts):
    c = lax.axis_index("c")
    s = lax.axis_index("s")
    wid = s * NC + c
    pltpu.sync_copy(ei_hbm.at[1, wid, 0], dstv)
    pltpu.sync_copy(ew_hbm.at[wid, 0], ewv)

    zero16 = jnp.zeros((16,), jnp.float32)

    def zero_chunk(i, _):
        degv[pl.ds(i * 16, 16)] = zero16
        return 0

    lax.fori_loop(0, DEG_ROWS, zero_chunk, 0)

    def acc(i, _):
        d16 = dstv[pl.ds(i * 16, 16)]
        w16 = ewv[pl.ds(i * 16, 16)]
        plsc.addupdate_scatter(degv, [d16], w16)
        return 0

    lax.fori_loop(0, DEG_CHUNKS, acc, 0)

    pltpu.sync_copy(degv, shared_parts.at[s])
    plsc.subcore_barrier()

    for t in range(NS):
        pltpu.sync_copy(shared_parts.at[t, pl.ds(s * DEG_SLICE, DEG_SLICE)],
                        redv.at[t])

    def red(k, _):
        acc16 = redv[0, pl.ds(k * 16, 16)]
        for t in range(1, NS):
            acc16 = acc16 + redv[t, pl.ds(k * 16, 16)]
        outv[pl.ds(k * 16, 16)] = acc16
        return 0

    lax.fori_loop(0, DEG_SLICE // 16, red, 0)
    pltpu.sync_copy(outv, deg_hbm.at[c, 0, pl.ds(s * DEG_SLICE, DEG_SLICE)])


_sc_deg = pl.kernel(
    _deg_body,
    out_type=jax.ShapeDtypeStruct((NC, 1, DEG_PAD), jnp.float32),
    mesh=_MESH,
    scratch_types=[
        pltpu.VMEM((DEG_E_PER_TILE,), jnp.int32),
        pltpu.VMEM((DEG_E_PER_TILE,), jnp.float32),
        pltpu.VMEM((DEG_PAD,), jnp.float32),
        pltpu.VMEM((NS, DEG_SLICE), jnp.float32),
        pltpu.VMEM((DEG_SLICE,), jnp.float32),
        pltpu.VMEM_SHARED((NS, DEG_PAD), jnp.float32),
    ],
    compiler_params=_SC_PARAMS,
)


def _tc1_body(x_ref, w_ref, deg_ref, h_ref):
    deg = deg_ref[:, :1] + deg_ref[:, 1:2] + 1.0
    dinv = jnp.where(deg > 0, lax.rsqrt(deg), 0.0)
    h = lax.dot_general(x_ref[...], w_ref[...],
                        (((1,), (1,)), ((), ())),
                        preferred_element_type=jnp.float32)
    h_ref[...] = h * dinv


def _tc1(x, W, deg2):
    R = 2048
    return pl.pallas_call(
        _tc1_body,
        grid=(N_PAD // R,),
        in_specs=[
            pl.BlockSpec((R, D), lambda i: (i, 0)),
            pl.BlockSpec((D, D), lambda i: (0, 0)),
            pl.BlockSpec((R, 2), lambda i: (i, 0)),
        ],
        out_specs=pl.BlockSpec((R, D), lambda i: (i, 0)),
        out_shape=jax.ShapeDtypeStruct((N_PAD, D), jnp.float32),
    )(x, W, deg2)


def _prop_body(h_hbm, src_hbm, dst_hbm, ew_hbm, out_hbm,
               src_a, dst_a, ew_a, src_b, dst_b, ew_b,
               rows_0, rows_1, shared_o,
               ssem_a, ssem_b, rsem_0, rsem_1, wsem):
    c = lax.axis_index("c")
    s = lax.axis_index("s")
    rbase = s * ROWS_PER_TILE

    zero16 = jnp.zeros((16,), jnp.float32)
    stg = ((src_a, dst_a, ew_a, ssem_a), (src_b, dst_b, ew_b, ssem_b))
    rows = (rows_0, rows_1)
    rsems = (rsem_0, rsem_1)

    base = jnp.where(c == 0, 0, NCH0)
    nch = jnp.where(c == 0, NCH0, NCH1)

    def stage_async(q, bufs):
        sc_, dc_, ec_, sem_ = bufs
        pltpu.async_copy(src_hbm.at[s, base + q], sc_, sem_)
        pltpu.async_copy(dst_hbm.at[s, base + q], dc_, sem_)
        pltpu.async_copy(ew_hbm.at[s, base + q, 0], ec_, sem_)

    def stage_wait(bufs):
        sc_, dc_, ec_, sem_ = bufs
        pltpu.make_async_copy(src_hbm.at[s, 0], sc_, sem_).wait()
        pltpu.make_async_copy(dst_hbm.at[s, 0], dc_, sem_).wait()
        pltpu.make_async_copy(ew_hbm.at[s, 0, 0], ec_, sem_).wait()

    @pl.when(c == 0)
    def _():
        pltpu.sync_copy(h_hbm.at[pl.ds(rbase, ROWS_PER_TILE)],
                        shared_o.at[pl.ds(rbase, ROWS_PER_TILE)])

    @pl.when(c == 1)
    def _():
        def zrow(i, _):
            for f in range(D // 16):
                rows_0[i, pl.ds(f * 16, 16)] = zero16
            return 0

        lax.fori_loop(0, PROP_BATCH, zrow, 0)
        for z in range(ROWS_PER_TILE // PROP_BATCH):
            pltpu.sync_copy(rows_0,
                            shared_o.at[pl.ds(rbase + z * PROP_BATCH,
                                              PROP_BATCH)])

    plsc.subcore_barrier()

    pltpu.sync_copy(src_hbm.at[s, base], src_a)
    pltpu.sync_copy(dst_hbm.at[s, base], dst_a)
    pltpu.sync_copy(ew_hbm.at[s, base, 0], ew_a)
    stage_async(1, stg[1])
    pltpu.async_copy(h_hbm.at[src_a.at[0]], rows_0, rsem_0)

    def superpair(qq, _):
        for p in range(2):
            q = qq * 2 + p
            sc_, dc_, ec_, _sem = stg[p]
            nstg = stg[1 - p]
            for j in range(CHUNK):
                rb, rsem = rows[j % 2], rsems[j % 2]
                nb, nsem = rows[(j + 1) % 2], rsems[(j + 1) % 2]
                pltpu.make_async_copy(h_hbm.at[sc_.at[j]], rb, rsem).wait()
                if j == 0:
                    @pl.when(q > 0)
                    def _():
                        pltpu.make_async_copy(
                            nb, shared_o.at[dc_.at[0]], wsem).wait()
                else:
                    pltpu.make_async_copy(
                        nb, shared_o.at[dc_.at[j - 1]], wsem).wait()
                if j < CHUNK - 1:
                    pltpu.async_copy(h_hbm.at[sc_.at[j + 1]], nb, nsem)
                else:
                    @pl.when(q + 1 < nch)
                    def _():
                        stage_wait(nstg)
                        pltpu.async_copy(h_hbm.at[nstg[0].at[0]], nb, nsem)

                def scale(i, _):
                    r = i * 2
                    bc0 = plsc.load_gather(
                        ec_, [jnp.full((16,), j * PROP_BATCH + r, jnp.int32)])
                    bc1 = plsc.load_gather(
                        ec_, [jnp.full((16,), j * PROP_BATCH + r + 1,
                                       jnp.int32)])
                    for f in range(D // 16):
                        rb[r, pl.ds(f * 16, 16)] = (
                            rb[r, pl.ds(f * 16, 16)] * bc0)
                    for f in range(D // 16):
                        rb[r + 1, pl.ds(f * 16, 16)] = (
                            rb[r + 1, pl.ds(f * 16, 16)] * bc1)
                    return 0

                lax.fori_loop(0, PROP_BATCH // 2, scale, 0)
                pltpu.async_copy(rb, shared_o.at[dc_.at[j]], wsem, add=True)

            @pl.when(q + 2 < nch)
            def _():
                stage_async(q + 2, (sc_, dc_, ec_, _sem))

        return 0

    lax.fori_loop(0, nch // 2, superpair, 0)
    pltpu.make_async_copy(rows_1, shared_o.at[dst_a.at[0]], wsem).wait()
    plsc.subcore_barrier()
    pltpu.sync_copy(shared_o.at[pl.ds(rbase, ROWS_PER_TILE)],
                    out_hbm.at[c, pl.ds(rbase, ROWS_PER_TILE)])


_sc_prop = pl.kernel(
    _prop_body,
    out_type=jax.ShapeDtypeStruct((NC, N_PAD, D), jnp.float32),
    mesh=_MESH,
    scratch_types=[
        pltpu.VMEM((CHUNK, PROP_BATCH), jnp.int32),
        pltpu.VMEM((CHUNK, PROP_BATCH), jnp.int32),
        pltpu.VMEM((CHUNK * PROP_BATCH,), jnp.float32),
        pltpu.VMEM((CHUNK, PROP_BATCH), jnp.int32),
        pltpu.VMEM((CHUNK, PROP_BATCH), jnp.int32),
        pltpu.VMEM((CHUNK * PROP_BATCH,), jnp.float32),
        pltpu.VMEM((PROP_BATCH, D), jnp.float32),
        pltpu.VMEM((PROP_BATCH, D), jnp.float32),
        pltpu.VMEM_SHARED((N_PAD, D), jnp.float32),
        pltpu.SemaphoreType.DMA,
        pltpu.SemaphoreType.DMA,
        pltpu.SemaphoreType.DMA,
        pltpu.SemaphoreType.DMA,
        pltpu.SemaphoreType.DMA,
    ],
    compiler_params=_SC_PARAMS,
)


def _tc2_body(o_ref, deg_ref, b_ref, ls_ref, lb_ref, out_ref):
    deg = deg_ref[:, :1] + deg_ref[:, 1:2] + 1.0
    dinv = jnp.where(deg > 0, lax.rsqrt(deg), 0.0)
    o = o_ref[0] + o_ref[1]
    o = o * dinv + b_ref[...]
    mean = jnp.mean(o, axis=1, keepdims=True)
    cent = o - mean
    var = jnp.mean(cent * cent, axis=1, keepdims=True)
    o = cent * lax.rsqrt(var + 1e-5) * ls_ref[...] + lb_ref[...]
    out_ref[...] = jnp.maximum(o, 0.0)


def _tc2(o2, deg2, b, ls, lb):
    R = 2048
    return pl.pallas_call(
        _tc2_body,
        grid=(N_PAD // R,),
        in_specs=[
            pl.BlockSpec((2, R, D), lambda i: (0, i, 0)),
            pl.BlockSpec((R, 2), lambda i: (i, 0)),
            pl.BlockSpec((1, D), lambda i: (0, 0)),
            pl.BlockSpec((1, D), lambda i: (0, 0)),
            pl.BlockSpec((1, D), lambda i: (0, 0)),
        ],
        out_specs=pl.BlockSpec((R, D), lambda i: (i, 0)),
        out_shape=jax.ShapeDtypeStruct((N_PAD, D), jnp.float32),
    )(o2, deg2, b, ls, lb)


@jax.jit
def kernel(x, edge_index, edge_weight, W, b, ln_scale, ln_bias):
    ei = edge_index.astype(jnp.int32)
    src = ei[0]
    dst = ei[1]
    ew = edge_weight.astype(jnp.float32)

    ei4 = ei.reshape(2, NW, 1, DEG_E_PER_TILE)
    ew1 = ew.reshape(NW, 1, DEG_E_PER_TILE)
    deg_parts = _sc_deg(ei4, ew1)
    deg2 = deg_parts.reshape(NC, DEG_PAD).T

    xp = jnp.pad(x, ((0, N_PAD - N_NODES), (0, 0)))
    h = _tc1(xp, W, deg2)

    pad = E_PAD - N_EDGES
    zi = jnp.zeros((pad,), jnp.int32)
    zf = jnp.zeros((pad,), jnp.float32)
    srcp = jnp.concatenate([src, zi]).reshape(NS, TOTC, CHUNK, PROP_BATCH)
    dstp = jnp.concatenate([dst, zi]).reshape(NS, TOTC, CHUNK, PROP_BATCH)
    ewp = jnp.concatenate([ew, zf]).reshape(NS, TOTC, 1, CHUNK * PROP_BATCH)
    o2 = _sc_prop(h, srcp, dstp, ewp)

    out = _tc2(o2, deg2, b.reshape(1, D), ln_scale.reshape(1, D),
               ln_bias.reshape(1, D))
    return out[:N_NODES]


# speedup vs baseline: 1.0732x; 1.0092x over previous
import functools

import jax
import jax.numpy as jnp
from jax import lax
from jax.experimental import pallas as pl
from jax.experimental.pallas import tpu as pltpu
from jax.experimental.pallas import tpu_sc as plsc

N_NODES = 10000
N_PAD = 10240
N_EDGES = 320000
D = 128
H = 64
NC = 2
NS = 16
NW = NC * NS

DEG_E_PER_TILE = N_EDGES // NW
DEG_CHUNKS = DEG_E_PER_TILE // 16
DEG_PAD = 10240
DEG_ROWS = DEG_PAD // 16
DEG_RED_BATCHES = DEG_ROWS // 128

PROP_BATCH = 128
CHUNK = 4
TOTC = 40
NCH0 = 30
NCH1 = TOTC - NCH0
E_PAD = NS * TOTC * CHUNK * PROP_BATCH
ROWS_PER_TILE = N_PAD // NS

_MESH = plsc.VectorSubcoreMesh(core_axis_name="c", subcore_axis_name="s")
_SC_PARAMS = pltpu.CompilerParams(needs_layout_passes=False)


DEG_SLICE = DEG_PAD // NS


def _deg_body(ei_hbm, ew_hbm, deg_hbm, dstv, ewv, degv, redv, outv, shared_parts):
    c = lax.axis_index("c")
    s = lax.axis_index("s")
    wid = s * NC + c
    pltpu.sync_copy(ei_hbm.at[1, wid, 0], dstv)
    pltpu.sync_copy(ew_hbm.at[wid, 0], ewv)

    zero16 = jnp.zeros((16,), jnp.float32)

    def zero_chunk(i, _):
        degv[pl.ds(i * 16, 16)] = zero16
        return 0

    lax.fori_loop(0, DEG_ROWS, zero_chunk, 0)

    def acc(i, _):
        d16 = dstv[pl.ds(i * 16, 16)]
        w16 = ewv[pl.ds(i * 16, 16)]
        plsc.addupdate_scatter(degv, [d16], w16)
        return 0

    lax.fori_loop(0, DEG_CHUNKS, acc, 0)

    pltpu.sync_copy(degv, shared_parts.at[s])
    plsc.subcore_barrier()

    for t in range(NS):
        pltpu.sync_copy(shared_parts.at[t, pl.ds(s * DEG_SLICE, DEG_SLICE)],
                        redv.at[t])

    def red(k, _):
        acc16 = redv[0, pl.ds(k * 16, 16)]
        for t in range(1, NS):
            acc16 = acc16 + redv[t, pl.ds(k * 16, 16)]
        outv[pl.ds(k * 16, 16)] = acc16
        return 0

    lax.fori_loop(0, DEG_SLICE // 16, red, 0)
    pltpu.sync_copy(outv, deg_hbm.at[c, 0, pl.ds(s * DEG_SLICE, DEG_SLICE)])


_sc_deg = pl.kernel(
    _deg_body,
    out_type=jax.ShapeDtypeStruct((NC, 1, DEG_PAD), jnp.float32),
    mesh=_MESH,
    scratch_types=[
        pltpu.VMEM((DEG_E_PER_TILE,), jnp.int32),
        pltpu.VMEM((DEG_E_PER_TILE,), jnp.float32),
        pltpu.VMEM((DEG_PAD,), jnp.float32),
        pltpu.VMEM((NS, DEG_SLICE), jnp.float32),
        pltpu.VMEM((DEG_SLICE,), jnp.float32),
        pltpu.VMEM_SHARED((NS, DEG_PAD), jnp.float32),
    ],
    compiler_params=_SC_PARAMS,
)


def _tc1_body(x_ref, w_ref, deg_ref, h_ref):
    deg = deg_ref[:, :1] + deg_ref[:, 1:2] + 1.0
    dinv = jnp.where(deg > 0, lax.rsqrt(deg), 0.0)
    h = lax.dot_general(x_ref[...], w_ref[...],
                        (((1,), (1,)), ((), ())),
                        preferred_element_type=jnp.float32)
    h_ref[...] = h * dinv


def _tc1(x, W, deg2):
    R = 2048
    return pl.pallas_call(
        _tc1_body,
        grid=(N_PAD // R,),
        in_specs=[
            pl.BlockSpec((R, D), lambda i: (i, 0)),
            pl.BlockSpec((D, D), lambda i: (0, 0)),
            pl.BlockSpec((R, 2), lambda i: (i, 0)),
        ],
        out_specs=pl.BlockSpec((R, D), lambda i: (i, 0)),
        out_shape=jax.ShapeDtypeStruct((N_PAD, D), jnp.float32),
    )(x, W, deg2)


def _prop_body(h_hbm, src_hbm, dst_hbm, ew_hbm, out_hbm,
               src_a, dst_a, ew_a, src_b, dst_b, ew_b,
               rows_0, rows_1, shared_o,
               ssem_a, ssem_b, rsem_0, rsem_1, wsem):
    c = lax.axis_index("c")
    s = lax.axis_index("s")
    rbase = s * ROWS_PER_TILE

    zero16 = jnp.zeros((16,), jnp.float32)
    stg = ((src_a, dst_a, ew_a, ssem_a), (src_b, dst_b, ew_b, ssem_b))
    rows = (rows_0, rows_1)
    rsems = (rsem_0, rsem_1)

    base = jnp.where(c == 0, 0, NCH0)
    nch = jnp.where(c == 0, NCH0, NCH1)

    def stage_async(q, bufs):
        sc_, dc_, ec_, sem_ = bufs
        pltpu.async_copy(src_hbm.at[s, base + q], sc_, sem_)
        pltpu.async_copy(dst_hbm.at[s, base + q], dc_, sem_)
        pltpu.async_copy(ew_hbm.at[s, base + q, 0], ec_, sem_)

    def stage_wait(bufs):
        sc_, dc_, ec_, sem_ = bufs
        pltpu.make_async_copy(src_hbm.at[s, 0], sc_, sem_).wait()
        pltpu.make_async_copy(dst_hbm.at[s, 0], dc_, sem_).wait()
        pltpu.make_async_copy(ew_hbm.at[s, 0, 0], ec_, sem_).wait()

    @pl.when(c == 0)
    def _():
        pltpu.sync_copy(h_hbm.at[pl.ds(rbase, ROWS_PER_TILE)],
                        shared_o.at[pl.ds(rbase, ROWS_PER_TILE)])

    @pl.when(c == 1)
    def _():
        def zrow(i, _):
            for f in range(D // 16):
                rows_0[i, pl.ds(f * 16, 16)] = zero16
            return 0

        lax.fori_loop(0, PROP_BATCH, zrow, 0)
        for z in range(ROWS_PER_TILE // PROP_BATCH):
            pltpu.sync_copy(rows_0,
                            shared_o.at[pl.ds(rbase + z * PROP_BATCH,
                                              PROP_BATCH)])

    plsc.subcore_barrier()

    pltpu.sync_copy(src_hbm.at[s, base], src_a)
    pltpu.sync_copy(dst_hbm.at[s, base], dst_a)
    pltpu.sync_copy(ew_hbm.at[s, base, 0], ew_a)
    stage_async(1, stg[1])
    pltpu.async_copy(h_hbm.at[src_a.at[0]], rows_0, rsem_0)

    def superpair(qq, _):
        for p in range(2):
            q = qq * 2 + p
            sc_, dc_, ec_, _sem = stg[p]
            nstg = stg[1 - p]
            for j in range(CHUNK):
                rb, rsem = rows[j % 2], rsems[j % 2]
                nb, nsem = rows[(j + 1) % 2], rsems[(j + 1) % 2]
                pltpu.make_async_copy(h_hbm.at[sc_.at[j]], rb, rsem).wait()
                if j == 0:
                    @pl.when(q > 0)
                    def _():
                        pltpu.make_async_copy(
                            nb, shared_o.at[dc_.at[0]], wsem).wait()
                else:
                    pltpu.make_async_copy(
                        nb, shared_o.at[dc_.at[j - 1]], wsem).wait()
                if j < CHUNK - 1:
                    pltpu.async_copy(h_hbm.at[sc_.at[j + 1]], nb, nsem)
                else:
                    @pl.when(q + 1 < nch)
                    def _():
                        stage_wait(nstg)
                        pltpu.async_copy(h_hbm.at[nstg[0].at[0]], nb, nsem)

                def scale(i, _):
                    r = i * 2
                    bc0 = plsc.load_gather(
                        ec_, [jnp.full((16,), j * PROP_BATCH + r, jnp.int32)])
                    bc1 = plsc.load_gather(
                        ec_, [jnp.full((16,), j * PROP_BATCH + r + 1,
                                       jnp.int32)])
                    for f in range(D // 16):
                        rb[r, pl.ds(f * 16, 16)] = (
                            rb[r, pl.ds(f * 16, 16)] * bc0)
                    for f in range(D // 16):
                        rb[r + 1, pl.ds(f * 16, 16)] = (
                            rb[r + 1, pl.ds(f * 16, 16)] * bc1)
                    return 0

                lax.fori_loop(0, PROP_BATCH // 2, scale, 0)
                pltpu.async_copy(rb, shared_o.at[dc_.at[j]], wsem, add=True)

            @pl.when(q + 2 < nch)
            def _():
                stage_async(q + 2, (sc_, dc_, ec_, _sem))

        return 0

    lax.fori_loop(0, nch // 2, superpair, 0)
    pltpu.make_async_copy(rows_1, shared_o.at[dst_a.at[0]], wsem).wait()
    plsc.subcore_barrier()
    pltpu.sync_copy(shared_o.at[pl.ds(rbase, ROWS_PER_TILE)],
                    out_hbm.at[c, pl.ds(rbase, ROWS_PER_TILE)])


_sc_prop = pl.kernel(
    _prop_body,
    out_type=jax.ShapeDtypeStruct((NC, N_PAD, D), jnp.float32),
    mesh=_MESH,
    scratch_types=[
        pltpu.VMEM((CHUNK, PROP_BATCH), jnp.int32),
        pltpu.VMEM((CHUNK, PROP_BATCH), jnp.int32),
        pltpu.VMEM((CHUNK * PROP_BATCH,), jnp.float32),
        pltpu.VMEM((CHUNK, PROP_BATCH), jnp.int32),
        pltpu.VMEM((CHUNK, PROP_BATCH), jnp.int32),
        pltpu.VMEM((CHUNK * PROP_BATCH,), jnp.float32),
        pltpu.VMEM((PROP_BATCH, D), jnp.float32),
        pltpu.VMEM((PROP_BATCH, D), jnp.float32),
        pltpu.VMEM_SHARED((N_PAD, D), jnp.float32),
        pltpu.SemaphoreType.DMA,
        pltpu.SemaphoreType.DMA,
        pltpu.SemaphoreType.DMA,
        pltpu.SemaphoreType.DMA,
        pltpu.SemaphoreType.DMA,
    ],
    compiler_params=_SC_PARAMS,
)


def _tc2_body(o_ref, deg_ref, b_ref, ls_ref, lb_ref, out_ref):
    deg = deg_ref[:, :1] + deg_ref[:, 1:2] + 1.0
    dinv = jnp.where(deg > 0, lax.rsqrt(deg), 0.0)
    o = o_ref[0] + o_ref[1]
    o = o * dinv + b_ref[...]
    mean = jnp.mean(o, axis=1, keepdims=True)
    cent = o - mean
    var = jnp.mean(cent * cent, axis=1, keepdims=True)
    o = cent * lax.rsqrt(var + 1e-5) * ls_ref[...] + lb_ref[...]
    out_ref[...] = jnp.maximum(o, 0.0)


def _tc2(o2, deg2, b, ls, lb):
    R = 2000
    return pl.pallas_call(
        _tc2_body,
        grid=(N_NODES // R,),
        in_specs=[
            pl.BlockSpec((2, R, D), lambda i: (0, i, 0)),
            pl.BlockSpec((R, 2), lambda i: (i, 0)),
            pl.BlockSpec((1, D), lambda i: (0, 0)),
            pl.BlockSpec((1, D), lambda i: (0, 0)),
            pl.BlockSpec((1, D), lambda i: (0, 0)),
        ],
        out_specs=pl.BlockSpec((R, D), lambda i: (i, 0)),
        out_shape=jax.ShapeDtypeStruct((N_NODES, D), jnp.float32),
    )(o2, deg2, b, ls, lb)


@jax.jit
def kernel(x, edge_index, edge_weight, W, b, ln_scale, ln_bias):
    ei = edge_index.astype(jnp.int32)
    src = ei[0]
    dst = ei[1]
    ew = edge_weight.astype(jnp.float32)

    ei4 = ei.reshape(2, NW, 1, DEG_E_PER_TILE)
    ew1 = ew.reshape(NW, 1, DEG_E_PER_TILE)
    deg_parts = _sc_deg(ei4, ew1)
    deg2 = deg_parts.reshape(NC, DEG_PAD).T

    xp = jnp.pad(x, ((0, N_PAD - N_NODES), (0, 0)))
    h = _tc1(xp, W, deg2)

    pad = E_PAD - N_EDGES
    zi = jnp.zeros((pad,), jnp.int32)
    zf = jnp.zeros((pad,), jnp.float32)
    srcp = jnp.concatenate([src, zi]).reshape(NS, TOTC, CHUNK, PROP_BATCH)
    dstp = jnp.concatenate([dst, zi]).reshape(NS, TOTC, CHUNK, PROP_BATCH)
    ewp = jnp.concatenate([ew, zf]).reshape(NS, TOTC, 1, CHUNK * PROP_BATCH)
    o2 = _sc_prop(h, srcp, dstp, ewp)

    return _tc2(o2, deg2, b.reshape(1, D), ln_scale.reshape(1, D),
                ln_bias.reshape(1, D))
